# baseline (device time: 19915 ns/iter reference)
import jax
import jax.numpy as jnp
from jax import lax
from jax.experimental import pallas as pl
from jax.experimental.pallas import tpu as pltpu

N_DEV = 4
B, Sq, Skv, Dh = 2, 256, 256, 64
H_LOC = 4
D_MODEL = 512
BLK = 64
HALF = B * Sq // 2


def kernel(x, Wq, K_ext, V_ext, Wo):
    my = lax.axis_index("i")
    Kh = lax.dynamic_slice_in_dim(K_ext, my * H_LOC, H_LOC, axis=2)
    Vh = lax.dynamic_slice_in_dim(V_ext, my * H_LOC, H_LOC, axis=2)
    x2d = x.reshape(B * Sq, D_MODEL)

    def body(x_ref, wq_ref, k_ref, v_ref, wo_ref, out_ref,
             ctx_ref, sbuf_ref, rbuf_ref, send_sems, recv_sems):
        pos = lax.axis_index("i")
        nb1 = jnp.bitwise_xor(pos, 1)
        nb2 = 3 - pos

        barrier_sem = pltpu.get_barrier_semaphore()
        for nbr in (nb1, nb2):
            pl.semaphore_signal(
                barrier_sem, inc=1,
                device_id=(nbr,), device_id_type=pl.DeviceIdType.MESH,
            )
        pl.semaphore_wait(barrier_sem, 2)

        q2d = jnp.dot(x_ref[:, :], wq_ref[:, :],
                      preferred_element_type=jnp.float32) * 0.125

        qb = lax.broadcasted_iota(jnp.int32, (Sq, Skv), 0) // BLK
        kb = lax.broadcasted_iota(jnp.int32, (Sq, Skv), 1) // BLK
        mask = (qb == kb) | (kb == 0) | ((qb + kb) % 3 == 0)

        def attn_head(b, h):
            qbh = q2d[b * Sq:(b + 1) * Sq, h * Dh:(h + 1) * Dh]
            s = lax.dot_general(
                qbh, k_ref[b, :, h, :], (((1,), (1,)), ((), ())),
                preferred_element_type=jnp.float32,
            )
            w = jnp.exp(jnp.where(mask, s, -1e9))
            w = w * (1.0 / jnp.sum(w, axis=1, keepdims=True))
            ctx_ref[b * Sq:(b + 1) * Sq, h * Dh:(h + 1) * Dh] = jnp.dot(
                w, v_ref[b, :, h, :], preferred_element_type=jnp.float32)

        def attn_batch(b):
            for h in range(H_LOC):
                attn_head(b, h)
            out_ref[b * Sq:(b + 1) * Sq, :] = jnp.dot(
                ctx_ref[b * Sq:(b + 1) * Sq, :], wo_ref[:, :],
                preferred_element_type=jnp.float32)

        QC = D_MODEL // 2

        def xchg(idx, half, dev0):
            rows = pl.ds(half * HALF, HALF)
            rdmas = []
            for q, dev in ((0, dev0), (1, nb1 + nb2 - dev0)):
                sbuf_ref[idx + q] = out_ref[rows, q * QC:(q + 1) * QC].astype(
                    jnp.bfloat16)
                rdma = pltpu.make_async_remote_copy(
                    src_ref=sbuf_ref.at[idx + q],
                    dst_ref=rbuf_ref.at[idx + q],
                    send_sem=send_sems.at[idx + q],
                    recv_sem=recv_sems.at[idx + q],
                    device_id=(dev,),
                    device_id_type=pl.DeviceIdType.MESH,
                )
                rdma.start()
                rdmas.append(rdma)
            return rdmas

        def absorb(rdmas, idx, half):
            rows = pl.ds(half * HALF, HALF)
            for q, rdma in enumerate(rdmas):
                rdma.wait()
                out_ref[rows, q * QC:(q + 1) * QC] = (
                    out_ref[rows, q * QC:(q + 1) * QC]
                    + rbuf_ref[idx + q].astype(jnp.float32))

        attn_batch(0)
        A1 = xchg(0, 0, nb1)
        attn_head(1, 0)
        attn_head(1, 1)
        absorb(A1, 0, 0)
        A2 = xchg(4, 0, nb2)
        attn_head(1, 2)
        attn_head(1, 3)
        out_ref[Sq:, :] = jnp.dot(
            ctx_ref[Sq:, :], wo_ref[:, :],
            preferred_element_type=jnp.float32)
        B1 = xchg(2, 1, nb2)
        absorb(A2, 4, 0)
        absorb(B1, 2, 1)
        B2 = xchg(6, 1, nb1)
        absorb(B2, 6, 1)

    out2d = pl.pallas_call(
        body,
        out_shape=jax.ShapeDtypeStruct((B * Sq, D_MODEL), jnp.float32),
        in_specs=[pl.BlockSpec(memory_space=pltpu.VMEM)] * 5,
        out_specs=pl.BlockSpec(memory_space=pltpu.VMEM),
        scratch_shapes=[
            pltpu.VMEM((B * Sq, H_LOC * Dh), jnp.float32),
            pltpu.VMEM((8, HALF, D_MODEL // 2), jnp.bfloat16),
            pltpu.VMEM((8, HALF, D_MODEL // 2), jnp.bfloat16),
            pltpu.SemaphoreType.DMA((8,)),
            pltpu.SemaphoreType.DMA((8,)),
        ],
        compiler_params=pltpu.CompilerParams(collective_id=0),
    )(x2d, Wq, Kh, Vh, Wo)
    return out2d.reshape(B, Sq, D_MODEL)


# device time: 16177 ns/iter; 1.2311x vs baseline; 1.2311x over previous
import jax
import jax.numpy as jnp
from jax import lax
from jax.experimental import pallas as pl
from jax.experimental.pallas import tpu as pltpu

N_DEV = 4
B, Sq, Skv, Dh = 2, 256, 256, 64
H_LOC = 4
D_MODEL = 512
BLK = 64
HALF = B * Sq // 2


def kernel(x, Wq, K_ext, V_ext, Wo):
    my = lax.axis_index("i")
    Kh = lax.dynamic_slice_in_dim(K_ext, my * H_LOC, H_LOC, axis=2)
    Vh = lax.dynamic_slice_in_dim(V_ext, my * H_LOC, H_LOC, axis=2)
    x2d = x.reshape(B * Sq, D_MODEL)

    def body(x_ref, wq_ref, k_ref, v_ref, wo_ref, out_ref,
             ctx_ref, sbuf_ref, rbuf_ref, send_sems, recv_sems):
        pos = lax.axis_index("i")
        nb1 = jnp.bitwise_xor(pos, 1)
        nb2 = 3 - pos

        barrier_sem = pltpu.get_barrier_semaphore()
        for nbr in (nb1, nb2):
            pl.semaphore_signal(
                barrier_sem, inc=1,
                device_id=(nbr,), device_id_type=pl.DeviceIdType.MESH,
            )

        q2d = jnp.dot(x_ref[:, :], wq_ref[:, :],
                      preferred_element_type=jnp.float32) * 0.125

        qb = lax.broadcasted_iota(jnp.int32, (Sq, Skv), 0) // BLK
        kb = lax.broadcasted_iota(jnp.int32, (Sq, Skv), 1) // BLK
        mask = (qb == kb) | (kb == 0) | ((qb + kb) % 3 == 0)

        def attn_head(b, h):
            qbh = q2d[b * Sq:(b + 1) * Sq, h * Dh:(h + 1) * Dh]
            s = lax.dot_general(
                qbh, k_ref[b, :, h, :], (((1,), (1,)), ((), ())),
                preferred_element_type=jnp.float32,
            )
            w = jnp.exp(jnp.where(mask, s, -1e9))
            w = w * (1.0 / jnp.sum(w, axis=1, keepdims=True))
            ctx_ref[b * Sq:(b + 1) * Sq, h * Dh:(h + 1) * Dh] = jnp.dot(
                w, v_ref[b, :, h, :], preferred_element_type=jnp.float32)

        def attn_batch(b):
            for h in range(H_LOC):
                attn_head(b, h)
            out_ref[b * Sq:(b + 1) * Sq, :] = jnp.dot(
                ctx_ref[b * Sq:(b + 1) * Sq, :], wo_ref[:, :],
                preferred_element_type=jnp.float32)

        QC = D_MODEL // 2

        def xchg(idx, half, dev0):
            rows = pl.ds(half * HALF, HALF)
            rdmas = []
            for q, dev in ((0, dev0), (1, nb1 + nb2 - dev0)):
                sbuf_ref[idx + q] = out_ref[rows, q * QC:(q + 1) * QC].astype(
                    jnp.bfloat16)
                rdma = pltpu.make_async_remote_copy(
                    src_ref=sbuf_ref.at[idx + q],
                    dst_ref=rbuf_ref.at[idx + q],
                    send_sem=send_sems.at[idx + q],
                    recv_sem=recv_sems.at[idx + q],
                    device_id=(dev,),
                    device_id_type=pl.DeviceIdType.MESH,
                )
                rdma.start()
                rdmas.append(rdma)
            return rdmas

        def absorb(rdmas, idx, half):
            rows = pl.ds(half * HALF, HALF)
            for q, rdma in enumerate(rdmas):
                rdma.wait()
                out_ref[rows, q * QC:(q + 1) * QC] = (
                    out_ref[rows, q * QC:(q + 1) * QC]
                    + rbuf_ref[idx + q].astype(jnp.float32))

        attn_batch(0)
        pl.semaphore_wait(barrier_sem, 2)
        A1 = xchg(0, 0, nb1)
        attn_batch(1)
        B1 = xchg(2, 1, nb2)
        absorb(A1, 0, 0)
        A2 = xchg(4, 0, nb2)
        absorb(B1, 2, 1)
        B2 = xchg(6, 1, nb1)
        absorb(A2, 4, 0)
        absorb(B2, 6, 1)

    out2d = pl.pallas_call(
        body,
        out_shape=jax.ShapeDtypeStruct((B * Sq, D_MODEL), jnp.float32),
        in_specs=[pl.BlockSpec(memory_space=pltpu.VMEM)] * 5,
        out_specs=pl.BlockSpec(memory_space=pltpu.VMEM),
        scratch_shapes=[
            pltpu.VMEM((B * Sq, H_LOC * Dh), jnp.float32),
            pltpu.VMEM((8, HALF, D_MODEL // 2), jnp.bfloat16),
            pltpu.VMEM((8, HALF, D_MODEL // 2), jnp.bfloat16),
            pltpu.SemaphoreType.DMA((8,)),
            pltpu.SemaphoreType.DMA((8,)),
        ],
        compiler_params=pltpu.CompilerParams(collective_id=0),
    )(x2d, Wq, Kh, Vh, Wo)
    return out2d.reshape(B, Sq, D_MODEL)


# device time: 14834 ns/iter; 1.3425x vs baseline; 1.0905x over previous
import jax
import jax.numpy as jnp
from jax import lax
from jax.experimental import pallas as pl
from jax.experimental.pallas import tpu as pltpu

N_DEV = 4
B, Sq, Skv, Dh = 2, 256, 256, 64
H_LOC = 4
D_MODEL = 512
BLK = 64
HALF = B * Sq // 2


def kernel(x, Wq, K_ext, V_ext, Wo):
    my = lax.axis_index("i")
    Kh = lax.dynamic_slice_in_dim(
        K_ext.reshape(B, Skv, 16 * Dh), my * H_LOC * Dh, H_LOC * Dh, axis=2)
    Vh = lax.dynamic_slice_in_dim(
        V_ext.reshape(B, Skv, 16 * Dh), my * H_LOC * Dh, H_LOC * Dh, axis=2)
    x2d = x.reshape(B * Sq, D_MODEL)

    def body(x_ref, wq_ref, k_ref, v_ref, wo_ref, out_ref,
             ctx_ref, sbuf_ref, rbuf_ref, send_sems, recv_sems):
        pos = lax.axis_index("i")
        nb1 = jnp.bitwise_xor(pos, 1)
        nb2 = 3 - pos

        barrier_sem = pltpu.get_barrier_semaphore()
        for nbr in (nb1, nb2):
            pl.semaphore_signal(
                barrier_sem, inc=1,
                device_id=(nbr,), device_id_type=pl.DeviceIdType.MESH,
            )

        q2d = jnp.dot(x_ref[:, :], wq_ref[:, :],
                      preferred_element_type=jnp.float32) * 0.125

        qb = lax.broadcasted_iota(jnp.int32, (Sq, Skv), 0) // BLK
        kb = lax.broadcasted_iota(jnp.int32, (Sq, Skv), 1) // BLK
        mask = (qb == kb) | (kb == 0) | ((qb + kb) % 3 == 0)

        def attn_head(b, h):
            qbh = q2d[b * Sq:(b + 1) * Sq, h * Dh:(h + 1) * Dh]
            s = lax.dot_general(
                qbh, k_ref[b, :, h * Dh:(h + 1) * Dh],
                (((1,), (1,)), ((), ())),
                preferred_element_type=jnp.float32,
            )
            w = jnp.exp(jnp.where(mask, s, -1e9))
            w = w * (1.0 / jnp.sum(w, axis=1, keepdims=True))
            ctx_ref[b * Sq:(b + 1) * Sq, h * Dh:(h + 1) * Dh] = jnp.dot(
                w, v_ref[b, :, h * Dh:(h + 1) * Dh],
                preferred_element_type=jnp.float32)

        def attn_batch(b):
            for h in range(H_LOC):
                attn_head(b, h)
            out_ref[b * Sq:(b + 1) * Sq, :] = jnp.dot(
                ctx_ref[b * Sq:(b + 1) * Sq, :], wo_ref[:, :],
                preferred_element_type=jnp.float32)

        QC = D_MODEL // 2

        def xchg(idx, half, dev0):
            rows = pl.ds(half * HALF, HALF)
            rdmas = []
            for q, dev in ((0, dev0), (1, nb1 + nb2 - dev0)):
                sbuf_ref[idx + q] = out_ref[rows, q * QC:(q + 1) * QC].astype(
                    jnp.bfloat16)
                rdma = pltpu.make_async_remote_copy(
                    src_ref=sbuf_ref.at[idx + q],
                    dst_ref=rbuf_ref.at[idx + q],
                    send_sem=send_sems.at[idx + q],
                    recv_sem=recv_sems.at[idx + q],
                    device_id=(dev,),
                    device_id_type=pl.DeviceIdType.MESH,
                )
                rdma.start()
                rdmas.append(rdma)
            return rdmas

        def absorb(rdmas, idx, half):
            rows = pl.ds(half * HALF, HALF)
            for q, rdma in enumerate(rdmas):
                rdma.wait()
                out_ref[rows, q * QC:(q + 1) * QC] = (
                    out_ref[rows, q * QC:(q + 1) * QC]
                    + rbuf_ref[idx + q].astype(jnp.float32))

        attn_batch(0)
        pl.semaphore_wait(barrier_sem, 2)
        A1 = xchg(0, 0, nb1)
        attn_batch(1)
        B1 = xchg(2, 1, nb2)
        absorb(A1, 0, 0)
        A2 = xchg(4, 0, nb2)
        absorb(B1, 2, 1)
        B2 = xchg(6, 1, nb1)
        absorb(A2, 4, 0)
        absorb(B2, 6, 1)

    out2d = pl.pallas_call(
        body,
        out_shape=jax.ShapeDtypeStruct((B * Sq, D_MODEL), jnp.float32),
        in_specs=[pl.BlockSpec(memory_space=pltpu.VMEM)] * 5,
        out_specs=pl.BlockSpec(memory_space=pltpu.VMEM),
        scratch_shapes=[
            pltpu.VMEM((B * Sq, H_LOC * Dh), jnp.float32),
            pltpu.VMEM((8, HALF, D_MODEL // 2), jnp.bfloat16),
            pltpu.VMEM((8, HALF, D_MODEL // 2), jnp.bfloat16),
            pltpu.SemaphoreType.DMA((8,)),
            pltpu.SemaphoreType.DMA((8,)),
        ],
        compiler_params=pltpu.CompilerParams(collective_id=0),
    )(x2d, Wq, Kh, Vh, Wo)
    return out2d.reshape(B, Sq, D_MODEL)


# device time: 14751 ns/iter; 1.3501x vs baseline; 1.0056x over previous
import jax
import jax.numpy as jnp
from jax import lax
from jax.experimental import pallas as pl
from jax.experimental.pallas import tpu as pltpu

N_DEV = 4
B, Sq, Skv, Dh = 2, 256, 256, 64
H_LOC = 4
D_MODEL = 512
BLK = 64
HALF = B * Sq // 2


def kernel(x, Wq, K_ext, V_ext, Wo):
    my = lax.axis_index("i")
    Kh = lax.dynamic_slice_in_dim(
        K_ext, my * H_LOC, H_LOC, axis=2).reshape(B, Skv, H_LOC * Dh)
    Vh = lax.dynamic_slice_in_dim(
        V_ext, my * H_LOC, H_LOC, axis=2).reshape(B, Skv, H_LOC * Dh)
    x2d = x.reshape(B * Sq, D_MODEL)

    def body(x_ref, wq_ref, k_ref, v_ref, wo_ref, out_ref,
             ctx_ref, sbuf_ref, rbuf_ref, send_sems, recv_sems):
        pos = lax.axis_index("i")
        nb1 = jnp.bitwise_xor(pos, 1)
        nb2 = 3 - pos

        barrier_sem = pltpu.get_barrier_semaphore()
        for nbr in (nb1, nb2):
            pl.semaphore_signal(
                barrier_sem, inc=1,
                device_id=(nbr,), device_id_type=pl.DeviceIdType.MESH,
            )

        q2d = jnp.dot(x_ref[:, :], wq_ref[:, :],
                      preferred_element_type=jnp.float32) * 0.125

        qb = lax.broadcasted_iota(jnp.int32, (Sq, Skv), 0) // BLK
        kb = lax.broadcasted_iota(jnp.int32, (Sq, Skv), 1) // BLK
        mask = (qb == kb) | (kb == 0) | ((qb + kb) % 3 == 0)

        def attn_head(b, h):
            qbh = q2d[b * Sq:(b + 1) * Sq, h * Dh:(h + 1) * Dh]
            s = lax.dot_general(
                qbh, k_ref[b, :, h * Dh:(h + 1) * Dh],
                (((1,), (1,)), ((), ())),
                preferred_element_type=jnp.float32,
            )
            w = jnp.exp(jnp.where(mask, s, -1e9))
            w = w * (1.0 / jnp.sum(w, axis=1, keepdims=True))
            ctx_ref[b * Sq:(b + 1) * Sq, h * Dh:(h + 1) * Dh] = jnp.dot(
                w, v_ref[b, :, h * Dh:(h + 1) * Dh],
                preferred_element_type=jnp.float32)

        def attn_batch(b):
            for h in range(H_LOC):
                attn_head(b, h)
            out_ref[b * Sq:(b + 1) * Sq, :] = jnp.dot(
                ctx_ref[b * Sq:(b + 1) * Sq, :], wo_ref[:, :],
                preferred_element_type=jnp.float32)

        QC = D_MODEL // 2

        def xchg(idx, half, dev0):
            rows = pl.ds(half * HALF, HALF)
            rdmas = []
            for q, dev in ((0, dev0), (1, nb1 + nb2 - dev0)):
                sbuf_ref[idx + q] = out_ref[rows, q * QC:(q + 1) * QC].astype(
                    jnp.bfloat16)
                rdma = pltpu.make_async_remote_copy(
                    src_ref=sbuf_ref.at[idx + q],
                    dst_ref=rbuf_ref.at[idx + q],
                    send_sem=send_sems.at[idx + q],
                    recv_sem=recv_sems.at[idx + q],
                    device_id=(dev,),
                    device_id_type=pl.DeviceIdType.MESH,
                )
                rdma.start()
                rdmas.append(rdma)
            return rdmas

        def absorb(rdmas, idx, half):
            rows = pl.ds(half * HALF, HALF)
            for q, rdma in enumerate(rdmas):
                rdma.wait()
                out_ref[rows, q * QC:(q + 1) * QC] = (
                    out_ref[rows, q * QC:(q + 1) * QC]
                    + rbuf_ref[idx + q].astype(jnp.float32))

        attn_batch(0)
        pl.semaphore_wait(barrier_sem, 2)
        A1 = xchg(0, 0, nb1)
        attn_batch(1)
        B1 = xchg(2, 1, nb2)
        absorb(A1, 0, 0)
        A2 = xchg(4, 0, nb2)
        absorb(B1, 2, 1)
        B2 = xchg(6, 1, nb1)
        absorb(A2, 4, 0)
        absorb(B2, 6, 1)

    out2d = pl.pallas_call(
        body,
        out_shape=jax.ShapeDtypeStruct((B * Sq, D_MODEL), jnp.float32),
        in_specs=[pl.BlockSpec(memory_space=pltpu.VMEM)] * 5,
        out_specs=pl.BlockSpec(memory_space=pltpu.VMEM),
        scratch_shapes=[
            pltpu.VMEM((B * Sq, H_LOC * Dh), jnp.float32),
            pltpu.VMEM((8, HALF, D_MODEL // 2), jnp.bfloat16),
            pltpu.VMEM((8, HALF, D_MODEL // 2), jnp.bfloat16),
            pltpu.SemaphoreType.DMA((8,)),
            pltpu.SemaphoreType.DMA((8,)),
        ],
        compiler_params=pltpu.CompilerParams(collective_id=0),
    )(x2d, Wq, Kh, Vh, Wo)
    return out2d.reshape(B, Sq, D_MODEL)
